# baseline (speedup 1.0000x reference)
import functools

import jax
import jax.numpy as jnp
from jax import lax
from jax.experimental import pallas as pl
from jax.experimental.pallas import tpu as pltpu
from jax.experimental.pallas import tpu_sc as plsc

L = 16
TT = 128
QT = 32
FC = 128


@functools.partial(jax.jit, static_argnames=("sizes",))
def _router(x2, band_of, plane_of, *, sizes):
    n_rows, F = x2.shape
    B = 4
    T = n_rows // B
    n_cols = F // FC
    n_th = 32 // n_cols
    t_span = T // n_th
    n_chunks = t_span // TT

    out_type = tuple(
        jax.ShapeDtypeStruct((nb, B, T), jnp.float32) for nb in sizes)

    mesh = plsc.VectorSubcoreMesh(core_axis_name="c", subcore_axis_name="s")

    scratch = [
        pltpu.VMEM((B * QT, FC), jnp.float32),
        pltpu.VMEM((B * QT, FC), jnp.float32),
        pltpu.VMEM((FC, B, TT), jnp.float32),
        pltpu.VMEM((FC + L,), jnp.int32),
        pltpu.VMEM((L,), jnp.int32),
        pltpu.SemaphoreType.DMA,
        pltpu.SemaphoreType.DMA,
        pltpu.SemaphoreType.DMA,
    ]

    @functools.partial(
        pl.kernel, out_type=out_type, mesh=mesh, scratch_types=scratch,
        compiler_params=pltpu.CompilerParams(
            needs_layout_passes=False, use_tc_tiling_on_sc=True))
    def k(x_hbm, rawof_hbm, colmeta_hbm, o0_hbm, o1_hbm, o2_hbm,
          ibuf0, ibuf1, obuf, rawv, metav, in_sem0, in_sem1, out_sem):
        wid = lax.axis_index("s") * 2 + lax.axis_index("c")
        fc = wid // n_th
        th = wid % n_th
        t0w = th * t_span

        pltpu.sync_copy(rawof_hbm.at[pl.ds(fc * (FC + L), FC + L)], rawv)
        pltpu.sync_copy(colmeta_hbm.at[pl.ds(fc * L, L)], metav)

        ibufs = (ibuf0, ibuf1)
        in_sems = (in_sem0, in_sem1)
        outs = (o0_hbm, o1_hbm, o2_hbm)
        iota = lax.iota(jnp.int32, L)
        fvs = [iota + g * L for g in range(FC // L)]
        meta = metav[pl.ds(0, L)]
        n_b = [meta[i] for i in range(3)]
        p0_b = [meta[3 + i] for i in range(3)]
        s_b = [0, n_b[0], n_b[0] + n_b[1]]
        diag = [lax.rem(iota + d, L) for d in range(L)]

        def in_copies(c, q, s):
            return [
                pltpu.make_async_copy(
                    x_hbm.at[pl.ds(b * T + t0w + c * TT + q * QT, QT),
                             pl.ds(fc * FC, FC)],
                    ibufs[s].at[pl.ds(b * QT, QT), :],
                    in_sems[s])
                for b in range(B)
            ]

        def drain_out():
            pltpu.make_async_copy(
                o2_hbm.at[pl.ds(0, FC), :, pl.ds(0, TT)], obuf, out_sem
            ).wait()

        def scatter_quarter(q, s):
            buf = ibufs[s]

            @pl.loop(0, B * (QT // L))
            def _bt(bt):
                b = bt // (QT // L)
                tb = bt - b * (QT // L)
                rbase = b * QT + tb * L
                tq = q * QT + tb * L
                bsp = jnp.full((L,), b, jnp.int32)
                rsp = jnp.full((L,), rbase, jnp.int32)
                tsp = jnp.full((L,), tq, jnp.int32)
                prev = None
                for d in range(L):
                    rv = rsp + diag[d]
                    cur = (tsp + diag[d],
                           [plsc.load_gather(buf, [rv, fvs[g]])
                            for g in range(FC // L)])
                    if prev is not None:
                        tv, vs = prev
                        for g in range(FC // L):
                            plsc.store_scatter(obuf, [fvs[g], bsp, tv], vs[g])
                    prev = cur
                tv, vs = prev
                for g in range(FC // L):
                    plsc.store_scatter(obuf, [fvs[g], bsp, tv], vs[g])

        for cp in in_copies(0, 0, 0):
            cp.start()
        for cp in in_copies(0, 1, 1):
            cp.start()

        @pl.loop(0, n_chunks)
        def _chunk(c):
            for q in range(4):
                s = q & 1
                for cp in in_copies(c, q, s):
                    cp.wait()
                if q == 0:
                    @pl.when(c > 0)
                    def _d():
                        drain_out()
                scatter_quarter(q, s)
                if q < 2:
                    for cp in in_copies(c, q + 2, s):
                        cp.start()
                else:
                    @pl.when(c + 1 < n_chunks)
                    def _n():
                        for cp in in_copies(c + 1, q - 2, s):
                            cp.start()

            for b_id in range(3):
                @pl.loop(0, n_b[b_id])
                def _plane(i):
                    row = rawv[pl.ds(s_b[b_id] + i, L)][0]
                    pltpu.async_copy(
                        obuf.at[row],
                        outs[b_id].at[p0_b[b_id] + i, :,
                                      pl.ds(t0w + c * TT, TT)],
                        out_sem)

        drain_out()

    return k(x2, band_of, plane_of)


def kernel(x, void_dims, identity_dims, prime_dims):
    B, T, F = x.shape
    sizes = (void_dims.shape[0], identity_dims.shape[0], prime_dims.shape[0])
    n_cols = F // FC
    bands = (void_dims, identity_dims, prime_dims)

    band_of = jnp.zeros((F,), jnp.int32)
    plane_of = jnp.zeros((F,), jnp.int32)
    for b_id, idx in enumerate(bands):
        idx = idx.astype(jnp.int32)
        band_of = band_of.at[idx].set(b_id)
        plane_of = plane_of.at[idx].set(
            jnp.arange(idx.shape[0], dtype=jnp.int32))

    col_edges = jnp.arange(n_cols + 1, dtype=jnp.int32) * FC
    p0 = jnp.stack([
        jnp.searchsorted(idx.astype(jnp.int32), col_edges).astype(jnp.int32)
        for idx in bands])
    n = p0[:, 1:] - p0[:, :-1]
    s = jnp.concatenate(
        [jnp.zeros((1, n_cols), jnp.int32), jnp.cumsum(n[:2], axis=0)])
    col_of = jnp.arange(F, dtype=jnp.int32) // FC
    sortpos = s[band_of, col_of] + plane_of - p0[band_of, col_of]
    rawof = jnp.zeros((n_cols, FC + L), jnp.int32)
    rawof = rawof.at[col_of, sortpos].set(
        jnp.arange(F, dtype=jnp.int32) % FC)
    colmeta = jnp.zeros((n_cols, L), jnp.int32)
    colmeta = colmeta.at[:, 0:3].set(n.T)
    colmeta = colmeta.at[:, 3:6].set(p0[:, :-1].T)

    o0, o1, o2 = _router(
        x.reshape(B * T, F), rawof.reshape(n_cols * (FC + L)),
        colmeta.reshape(n_cols * L), sizes=sizes)
    return (o0.transpose(1, 2, 0),
            o1.transpose(1, 2, 0),
            o2.transpose(1, 2, 0))


# speedup vs baseline: 1.2487x; 1.2487x over previous
import functools

import jax
import jax.numpy as jnp
from jax import lax
from jax.experimental import pallas as pl
from jax.experimental.pallas import tpu as pltpu
from jax.experimental.pallas import tpu_sc as plsc

L = 16
TT = 128
QT = 32
FC = 128


@functools.partial(jax.jit, static_argnames=("sizes", "B"))
def _router(x2, band_of, plane_of, *, sizes, B):
    n_rows, F = x2.shape
    T = n_rows // B
    n_cols = F // FC
    n_th = 32 // n_cols
    t_span = T // n_th
    n_chunks = t_span // TT

    out_type = tuple(
        jax.ShapeDtypeStruct((nb, B, T), jnp.float32) for nb in sizes)

    mesh = plsc.VectorSubcoreMesh(core_axis_name="c", subcore_axis_name="s")

    scratch = [
        pltpu.VMEM((B * QT, FC), jnp.float32),
        pltpu.VMEM((B * QT, FC), jnp.float32),
        pltpu.VMEM((FC, B, TT), jnp.float32),
        pltpu.VMEM((FC,), jnp.int32),
        pltpu.VMEM((FC,), jnp.int32),
        pltpu.SemaphoreType.DMA,
        pltpu.SemaphoreType.DMA,
        pltpu.SemaphoreType.DMA,
    ]

    @functools.partial(
        pl.kernel, out_type=out_type, mesh=mesh, scratch_types=scratch,
        compiler_params=pltpu.CompilerParams(
            needs_layout_passes=False, use_tc_tiling_on_sc=True))
    def k(x_hbm, band_hbm, plane_hbm, o0_hbm, o1_hbm, o2_hbm,
          ibuf0, ibuf1, obuf, bandv, planev, in_sem0, in_sem1, out_sem):
        wid = lax.axis_index("s") * 2 + lax.axis_index("c")
        fc = wid // n_th
        th = wid % n_th
        t0w = th * t_span

        pltpu.sync_copy(band_hbm.at[pl.ds(fc * FC, FC)], bandv)
        pltpu.sync_copy(plane_hbm.at[pl.ds(fc * FC, FC)], planev)

        ibufs = (ibuf0, ibuf1)
        in_sems = (in_sem0, in_sem1)
        outs = (o0_hbm, o1_hbm, o2_hbm)
        iota = lax.iota(jnp.int32, L)
        fvs = [iota + g * L for g in range(FC // L)]
        diag = [lax.rem(iota + d, L) for d in range(L)]

        def in_copies(c, q, s):
            return [
                pltpu.make_async_copy(
                    x_hbm.at[pl.ds(b * T + t0w + c * TT + q * QT, QT),
                             pl.ds(fc * FC, FC)],
                    ibufs[s].at[pl.ds(b * QT, QT), :],
                    in_sems[s])
                for b in range(B)
            ]

        def drain_out():
            pltpu.make_async_copy(
                o2_hbm.at[pl.ds(0, FC), :, pl.ds(0, TT)], obuf, out_sem
            ).wait()

        def scatter_quarter(q, s):
            buf = ibufs[s]

            @pl.loop(0, B * (QT // L))
            def _bt(bt):
                b = bt // (QT // L)
                tb = bt - b * (QT // L)
                rbase = b * QT + tb * L
                tq = q * QT + tb * L
                bsp = jnp.full((L,), b, jnp.int32)
                rsp = jnp.full((L,), rbase, jnp.int32)
                tsp = jnp.full((L,), tq, jnp.int32)
                prev = None
                for d in range(L):
                    rv = rsp + diag[d]
                    cur = (tsp + diag[d],
                           [plsc.load_gather(buf, [rv, fvs[g]])
                            for g in range(FC // L)])
                    if prev is not None:
                        tv, vs = prev
                        for g in range(FC // L):
                            plsc.store_scatter(obuf, [fvs[g], bsp, tv], vs[g])
                    prev = cur
                tv, vs = prev
                for g in range(FC // L):
                    plsc.store_scatter(obuf, [fvs[g], bsp, tv], vs[g])

        for cp in in_copies(0, 0, 0):
            cp.start()
        for cp in in_copies(0, 1, 1):
            cp.start()

        @pl.loop(0, n_chunks)
        def _chunk(c):
            for q in range(4):
                s = q & 1
                for cp in in_copies(c, q, s):
                    cp.wait()
                if q == 0:
                    @pl.when(c > 0)
                    def _d():
                        drain_out()
                scatter_quarter(q, s)
                if q < 2:
                    for cp in in_copies(c, q + 2, s):
                        cp.start()
                else:
                    @pl.when(c + 1 < n_chunks)
                    def _n():
                        for cp in in_copies(c + 1, q - 2, s):
                            cp.start()

            @pl.loop(0, FC // L)
            def _plane(j):
                bdv = bandv[pl.ds(j * L, L)]
                pv = planev[pl.ds(j * L, L)]
                for l in range(L):
                    bd = bdv[l]
                    p = pv[l]
                    src = obuf.at[j * L + l]
                    for b_id in range(3):
                        @pl.when(bd == b_id)
                        def _w():
                            pltpu.async_copy(
                                src,
                                outs[b_id].at[p, :, pl.ds(t0w + c * TT, TT)],
                                out_sem)

        drain_out()

    return k(x2, band_of, plane_of)


def kernel(x, void_dims, identity_dims, prime_dims):
    B, T, F = x.shape
    sizes = (void_dims.shape[0], identity_dims.shape[0], prime_dims.shape[0])
    band_of = jnp.zeros((F,), jnp.int32)
    plane_of = jnp.zeros((F,), jnp.int32)
    for b_id, idx in enumerate((void_dims, identity_dims, prime_dims)):
        idx = idx.astype(jnp.int32)
        band_of = band_of.at[idx].set(b_id)
        plane_of = plane_of.at[idx].set(
            jnp.arange(idx.shape[0], dtype=jnp.int32))
    o0, o1, o2 = _router(
        x.reshape(B * T, F), band_of, plane_of, sizes=sizes, B=B)
    return (o0.transpose(1, 2, 0),
            o1.transpose(1, 2, 0),
            o2.transpose(1, 2, 0))
